# baseline (speedup 1.0000x reference)
import jax
import jax.numpy as jnp
from jax import lax
from jax.experimental import pallas as pl
from jax.experimental.pallas import tpu as pltpu

_B = 8192
_D = 2048
_K = 64
_BM = 512


def _fused(aux_ref, x_ref, u_ref, tb_ref, b1_ref, w2_ref, b2_ref,
           db1_ref, db2_ref, out_ref, dec_ref):
    @pl.when(pl.program_id(0) == 0)
    def _():
        t = tb_ref[...]
        h2 = jnp.dot(t, aux_ref[0:1024, 64:96],
                     preferred_element_type=jnp.float32)
        h2 = jnp.maximum(h2 + db1_ref[...], 0.0)
        dec_ref[...] = jnp.dot(h2, aux_ref[0:32, 96:97],
                               preferred_element_type=jnp.float32) + db2_ref[...]

    x = x_ref[...]
    h = jnp.dot(x.astype(jnp.bfloat16),
                aux_ref[0:2048, 0:64].astype(jnp.bfloat16),
                preferred_element_type=jnp.float32)
    h = jnp.maximum(h + b1_ref[...], 0.0)
    logits = jnp.dot(h, w2_ref[...],
                     preferred_element_type=jnp.float32) + b2_ref[...]
    boosted = logits + 1.0
    g = -jnp.log(-jnp.log(u_ref[...]))
    s = boosted + g

    m = jnp.max(s, axis=-1, keepdims=True)
    iota = lax.broadcasted_iota(jnp.int32, s.shape, 1)
    first = jnp.min(jnp.where(s == m, iota, _K), axis=-1, keepdims=True)
    onehot = (iota == first).astype(jnp.float32)

    out = jnp.dot(onehot, dec_ref[...],
                  preferred_element_type=jnp.float32)
    out_ref[...] = out.reshape(_BM)


def kernel(x, gumbel_u, thought_bank, sel_w1, sel_b1, sel_w2, sel_b2,
           dec_w1, dec_b1, dec_w2, dec_b2):
    aux = jnp.concatenate([
        sel_w1,
        jnp.pad(dec_w1, ((0, _D - 1024), (0, 0))),
        jnp.pad(dec_w2, ((0, _D - 32), (0, 31))),
    ], axis=1)
    grid = (_B // _BM,)
    out = pl.pallas_call(
        _fused,
        grid=grid,
        in_specs=[
            pl.BlockSpec((_D, 128), lambda i: (0, 0)),
            pl.BlockSpec((_BM, _D), lambda i: (i, 0)),
            pl.BlockSpec((_BM, _K), lambda i: (i, 0)),
            pl.BlockSpec((_K, 1024), lambda i: (0, 0)),
            pl.BlockSpec((1, _K), lambda i: (0, 0)),
            pl.BlockSpec((_K, _K), lambda i: (0, 0)),
            pl.BlockSpec((1, _K), lambda i: (0, 0)),
            pl.BlockSpec((1, 32), lambda i: (0, 0)),
            pl.BlockSpec((1, 1), lambda i: (0, 0)),
        ],
        out_specs=pl.BlockSpec((_BM,), lambda i: (i,)),
        out_shape=jax.ShapeDtypeStruct((_B,), jnp.float32),
        scratch_shapes=[pltpu.VMEM((_K, 1), jnp.float32)],
    )(aux, x, gumbel_u, thought_bank, sel_b1.reshape(1, _K), sel_w2,
      sel_b2.reshape(1, _K), dec_b1.reshape(1, 32), dec_b2.reshape(1, 1))
    return out


# speedup vs baseline: 1.1197x; 1.1197x over previous
import jax
import jax.numpy as jnp
from jax import lax
from jax.experimental import pallas as pl
from jax.experimental.pallas import tpu as pltpu

_B = 8192
_D = 2048
_K = 64
_BM = 1024


def _fused(aux_ref, x_ref, u_ref, tb_ref, b1_ref, w2_ref, b2_ref,
           db1_ref, db2_ref, out_ref, dec_ref):
    @pl.when(pl.program_id(0) == 0)
    def _():
        t = tb_ref[...]
        h2 = jnp.dot(t, aux_ref[0:1024, 64:96],
                     preferred_element_type=jnp.float32)
        h2 = jnp.maximum(h2 + db1_ref[...], 0.0)
        dec_ref[...] = jnp.dot(h2, aux_ref[0:32, 96:97],
                               preferred_element_type=jnp.float32) + db2_ref[...]

    x = x_ref[...]
    h = jnp.dot(x.astype(jnp.bfloat16),
                aux_ref[0:2048, 0:64].astype(jnp.bfloat16),
                preferred_element_type=jnp.float32)
    h = jnp.maximum(h + b1_ref[...], 0.0)
    logits = jnp.dot(h, w2_ref[...],
                     preferred_element_type=jnp.float32) + b2_ref[...]
    boosted = logits + 1.0
    g = -jnp.log(-jnp.log(u_ref[...]))
    s = boosted + g

    m = jnp.max(s, axis=-1, keepdims=True)
    iota = lax.broadcasted_iota(jnp.int32, s.shape, 1)
    first = jnp.min(jnp.where(s == m, iota, _K), axis=-1, keepdims=True)
    onehot = (iota == first).astype(jnp.float32)

    out = jnp.dot(onehot, dec_ref[...],
                  preferred_element_type=jnp.float32)
    out_ref[...] = out.reshape(_BM)


def kernel(x, gumbel_u, thought_bank, sel_w1, sel_b1, sel_w2, sel_b2,
           dec_w1, dec_b1, dec_w2, dec_b2):
    aux = jnp.concatenate([
        sel_w1,
        jnp.pad(dec_w1, ((0, _D - 1024), (0, 0))),
        jnp.pad(dec_w2, ((0, _D - 32), (0, 31))),
    ], axis=1)
    grid = (_B // _BM,)
    out = pl.pallas_call(
        _fused,
        grid=grid,
        in_specs=[
            pl.BlockSpec((_D, 128), lambda i: (0, 0)),
            pl.BlockSpec((_BM, _D), lambda i: (i, 0)),
            pl.BlockSpec((_BM, _K), lambda i: (i, 0)),
            pl.BlockSpec((_K, 1024), lambda i: (0, 0)),
            pl.BlockSpec((1, _K), lambda i: (0, 0)),
            pl.BlockSpec((_K, _K), lambda i: (0, 0)),
            pl.BlockSpec((1, _K), lambda i: (0, 0)),
            pl.BlockSpec((1, 32), lambda i: (0, 0)),
            pl.BlockSpec((1, 1), lambda i: (0, 0)),
        ],
        out_specs=pl.BlockSpec((_BM,), lambda i: (i,)),
        out_shape=jax.ShapeDtypeStruct((_B,), jnp.float32),
        scratch_shapes=[pltpu.VMEM((_K, 1), jnp.float32)],
    )(aux, x, gumbel_u, thought_bank, sel_b1.reshape(1, _K), sel_w2,
      sel_b2.reshape(1, _K), dec_b1.reshape(1, 32), dec_b2.reshape(1, 1))
    return out


# speedup vs baseline: 1.1514x; 1.0284x over previous
import jax
import jax.numpy as jnp
from jax import lax
from jax.experimental import pallas as pl
from jax.experimental.pallas import tpu as pltpu

_B = 8192
_D = 2048
_K = 64
_BM = 1024


def _fused(aux_ref, x_ref, u_ref, tb_ref, b1_ref, w2_ref, b2_ref,
           db1_ref, dw2_ref, db2_ref, out_ref, dec_ref):
    @pl.when(pl.program_id(0) == 0)
    def _():
        t = tb_ref[...]
        h2 = jnp.dot(t, aux_ref[0:1024, 64:96],
                     preferred_element_type=jnp.float32)
        h2 = jnp.maximum(h2 + db1_ref[...], 0.0)
        dec_ref[...] = (jnp.sum(h2 * dw2_ref[...], axis=1, keepdims=True)
                        + db2_ref[...])

    x = x_ref[...]
    h = jnp.dot(x.astype(jnp.bfloat16),
                aux_ref[0:2048, 0:64].astype(jnp.bfloat16),
                preferred_element_type=jnp.float32)
    h = jnp.maximum(h + b1_ref[...], 0.0)
    logits = jnp.dot(h, w2_ref[...],
                     preferred_element_type=jnp.float32) + b2_ref[...]
    boosted = logits + 1.0
    g = -jnp.log(-jnp.log(u_ref[...]))
    s = boosted + g

    m = jnp.max(s, axis=-1, keepdims=True)
    iota = lax.broadcasted_iota(jnp.int32, s.shape, 1)
    first = jnp.min(jnp.where(s == m, iota, _K), axis=-1, keepdims=True)
    onehot = (iota == first).astype(jnp.float32)

    out = jnp.dot(onehot, dec_ref[...],
                  preferred_element_type=jnp.float32)
    out_ref[...] = out.reshape(_BM)


def kernel(x, gumbel_u, thought_bank, sel_w1, sel_b1, sel_w2, sel_b2,
           dec_w1, dec_b1, dec_w2, dec_b2):
    aux = jnp.concatenate([
        sel_w1,
        jnp.pad(dec_w1, ((0, _D - 1024), (0, 32))),
    ], axis=1)
    grid = (_B // _BM,)
    out = pl.pallas_call(
        _fused,
        grid=grid,
        in_specs=[
            pl.BlockSpec((_D, 128), lambda i: (0, 0)),
            pl.BlockSpec((_BM, _D), lambda i: (i, 0)),
            pl.BlockSpec((_BM, _K), lambda i: (i, 0)),
            pl.BlockSpec((_K, 1024), lambda i: (0, 0)),
            pl.BlockSpec((1, _K), lambda i: (0, 0)),
            pl.BlockSpec((_K, _K), lambda i: (0, 0)),
            pl.BlockSpec((1, _K), lambda i: (0, 0)),
            pl.BlockSpec((1, 32), lambda i: (0, 0)),
            pl.BlockSpec((1, 32), lambda i: (0, 0)),
            pl.BlockSpec((1, 1), lambda i: (0, 0)),
        ],
        out_specs=pl.BlockSpec((_BM,), lambda i: (i,)),
        out_shape=jax.ShapeDtypeStruct((_B,), jnp.float32),
        scratch_shapes=[pltpu.VMEM((_K, 1), jnp.float32)],
    )(aux, x, gumbel_u, thought_bank, sel_b1.reshape(1, _K), sel_w2,
      sel_b2.reshape(1, _K), dec_b1.reshape(1, 32), dec_w2.reshape(1, 32),
      dec_b2.reshape(1, 1))
    return out
